# Initial kernel scaffold; baseline (speedup 1.0000x reference)
import jax
import jax.numpy as jnp
from jax.experimental import pallas as pl


def kernel(x_cat, tables):
    raise NotImplementedError("write your pallas kernel here")


# speedup vs baseline: 1.1108x; 1.1108x over previous
import functools

import jax
import jax.numpy as jnp
from jax import lax
from jax.experimental import pallas as pl
from jax.experimental.pallas import tpu as pltpu
from jax.experimental.pallas import tpu_sc as plsc

_NUM_FIELDS = 26
_VOCAB = 100000
_EMB = 32
_BATCH = 16384
_TOT = _BATCH * _NUM_FIELDS
_NW = 32
_PER_W = _TOT // _NW
_CH = 128
_NCH = _PER_W // _CH
_LANES = 16

_mesh = plsc.VectorSubcoreMesh(core_axis_name="c", subcore_axis_name="s")


@functools.partial(
    pl.kernel,
    mesh=_mesh,
    out_type=jax.ShapeDtypeStruct((_TOT, _EMB), jnp.float32),
    scratch_types=[
        pltpu.VMEM((_CH,), jnp.int32),
        pltpu.VMEM((_CH, _EMB), jnp.float32),
        pltpu.SemaphoreType.DMA,
    ],
    compiler_params=pltpu.CompilerParams(use_tc_tiling_on_sc=False),
)
def _sc_gather(xflat_hbm, table_hbm, out_hbm, idx_v, rows_v, sem):
    wid = lax.axis_index("s") * 2 + lax.axis_index("c")
    base = wid * _PER_W

    def chunk_body(g, carry):
        cb = base + g * _CH
        pltpu.sync_copy(xflat_hbm.at[pl.ds(cb, _CH)], idx_v)

        def vstep(t, c):
            j = cb + t * _LANES + lax.iota(jnp.int32, _LANES)
            f = lax.rem(j, _NUM_FIELDS)
            idx_v[pl.ds(t * _LANES, _LANES)] = (
                idx_v[pl.ds(t * _LANES, _LANES)] + f * _VOCAB)
            return c

        lax.fori_loop(0, _CH // _LANES, vstep, 0)
        pltpu.async_copy(table_hbm.at[idx_v], rows_v, sem).wait()
        pltpu.sync_copy(rows_v, out_hbm.at[pl.ds(cb, _CH)])
        return carry

    lax.fori_loop(0, _NCH, chunk_body, 0)


def kernel(x_cat, tables):
    xflat = x_cat.reshape(-1)
    tflat = tables.reshape(_NUM_FIELDS * _VOCAB, _EMB)
    out = _sc_gather(xflat, tflat)
    return out.reshape(_BATCH, _NUM_FIELDS * _EMB)


# speedup vs baseline: 1.2169x; 1.0955x over previous
import functools

import jax
import jax.numpy as jnp
from jax import lax
from jax.experimental import pallas as pl
from jax.experimental.pallas import tpu as pltpu
from jax.experimental.pallas import tpu_sc as plsc

_NUM_FIELDS = 26
_VOCAB = 100000
_EMB = 32
_BATCH = 16384
_TOT = _BATCH * _NUM_FIELDS
_NW = 32
_PER_W = _TOT // _NW
_CH = 128
_NCH = _PER_W // _CH
_K = 8
_GROUP = _K * _CH
_NG = _PER_W // _GROUP
_LANES = 16

_mesh = plsc.VectorSubcoreMesh(core_axis_name="c", subcore_axis_name="s")


@functools.partial(
    pl.kernel,
    mesh=_mesh,
    out_type=jax.ShapeDtypeStruct((_TOT, _EMB), jnp.float32),
    scratch_types=[
        pltpu.VMEM((_NCH, _CH), jnp.int32),
        pltpu.VMEM((2, _GROUP, _EMB), jnp.float32),
        pltpu.SemaphoreType.DMA,
        pltpu.SemaphoreType.DMA,
    ],
    compiler_params=pltpu.CompilerParams(use_tc_tiling_on_sc=False),
)
def _sc_gather(xflat_hbm, table_hbm, out_hbm, idx_v, rows_v, sem_g, sem_st):
    wid = lax.axis_index("s") * 2 + lax.axis_index("c")
    base = wid * _PER_W
    crow = wid * _NCH

    pltpu.sync_copy(xflat_hbm.at[pl.ds(crow, _NCH)], idx_v)

    def offset_group(g):
        def vstep(t, c):
            jj = g * _K + t // (_CH // _LANES)
            col = lax.rem(t, _CH // _LANES) * _LANES
            p = base + jj * _CH + col
            j = p + lax.iota(jnp.int32, _LANES)
            f = lax.rem(j, _NUM_FIELDS)
            idx_v[jj, pl.ds(col, _LANES)] = (
                idx_v[jj, pl.ds(col, _LANES)] + f * _VOCAB)
            return c

        lax.fori_loop(0, _K * (_CH // _LANES), vstep, 0)

    def fire_gathers(g, buf):
        return [
            pltpu.async_copy(
                table_hbm.at[idx_v.at[g * _K + j]],
                rows_v.at[buf, pl.ds(j * _CH, _CH)],
                sem_g)
            for j in range(_K)
        ]

    def fire_store(g, buf):
        return pltpu.async_copy(
            rows_v.at[buf],
            out_hbm.at[pl.ds(base + g * _GROUP, _GROUP)],
            sem_st)

    offset_group(0)
    g_descs = {0: fire_gathers(0, 0)}
    s_descs = {}
    for g in range(_NG):
        if g + 1 < _NG:
            offset_group(g + 1)
            if g >= 1:
                s_descs.pop(g - 1).wait()
            g_descs[g + 1] = fire_gathers(g + 1, (g + 1) % 2)
        for d in g_descs.pop(g):
            d.wait()
        s_descs[g] = fire_store(g, g % 2)
    s_descs.pop(_NG - 2).wait()
    s_descs.pop(_NG - 1).wait()


def kernel(x_cat, tables):
    xflat = x_cat.reshape(_TOT // _CH, _CH)
    tflat = tables.reshape(_NUM_FIELDS * _VOCAB, _EMB)
    out = _sc_gather(xflat, tflat)
    return out.reshape(_BATCH, _NUM_FIELDS * _EMB)
